# baseline (speedup 1.0000x reference)
import math
from functools import partial

import jax
import jax.numpy as jnp
import numpy as np
from jax.experimental import pallas as pl
from jax.experimental.pallas import tpu as pltpu

_MASK_PROB = 0.065
_MASK_LENGTH = 10

_I32 = jnp.int32
_SIGN = np.int32(-2147483648)


def _rotl(x, r):
    return jax.lax.shift_left(x, _I32(r)) | jax.lax.shift_right_logical(
        x, _I32(32 - r))


def _threefry2x32(k0, k1, x0, x1):
    ks0, ks1 = k0, k1
    ks2 = ks0 ^ ks1 ^ np.int32(0x1BD11BDA)
    rots = ((13, 15, 26, 6), (17, 29, 16, 24))
    sched = ((ks1, ks2), (ks2, ks0), (ks0, ks1), (ks1, ks2), (ks2, ks0))
    x0 = x0 + ks0
    x1 = x1 + ks1
    for i in range(5):
        for r in rots[i % 2]:
            x0 = x0 + x1
            x1 = _rotl(x1, r)
            x1 = x1 ^ x0
        a, b = sched[i]
        x0 = x0 + a
        x1 = x1 + b + np.int32(i + 1)
    return x0, x1


def _child_key(k0, k1, idx):
    return _threefry2x32(k0, k1, _I32(0), _I32(idx))


def _row_bits_sortable(k0, k1, lanes_iota, n_valid, pad_val):
    o0, o1 = _threefry2x32(k0, k1, jnp.zeros_like(lanes_iota), lanes_iota)
    bits = o0 ^ o1
    srt = bits ^ _SIGN
    return jnp.where(lanes_iota < _I32(n_valid), srt, jnp.full_like(lanes_iota, pad_val))


def _mask_kernel(o_ref, k1_scr, *, rows, cols, n_valid, n_take, num_rounds,
                 mask_len):
    b = pl.program_id(0)
    total = rows * cols
    flat_iota = (
        jax.lax.broadcasted_iota(_I32, (rows, cols), 0) * _I32(cols)
        + jax.lax.broadcasted_iota(_I32, (rows, cols), 1))
    imax = np.int32(2147483647)

    rk0, rk1 = _threefry2x32(_I32(0), _I32(42), _I32(0), b)
    round_sort_keys = []
    for _ in range(num_rounds):
        nk0, nk1 = _child_key(rk0, rk1, 0)
        sk0, sk1 = _child_key(rk0, rk1, 1)
        round_sort_keys.append(
            _row_bits_sortable(sk0, sk1, flat_iota, n_valid, imax))
        rk0, rk1 = nk0, nk1
    k1s = round_sort_keys[0]
    k2s = round_sort_keys[-1]

    def _descend(j, v):
        bit = jax.lax.shift_left(_I32(1), _I32(31) - j)
        try_pat = v | bit
        try_s = try_pat ^ _SIGN
        cnt_lt = jnp.sum(jnp.where(k2s < try_s, jnp.float32(1), jnp.float32(0)))
        return jnp.where(cnt_lt >= jnp.float32(n_take), v, try_pat)

    v54_pat = jax.lax.fori_loop(0, 32, _descend, _I32(0))
    v54_s = v54_pat ^ _SIGN
    p_ind = jnp.where(k2s <= v54_s, jnp.float32(1), jnp.float32(0))

    kb = k1s[:, :, None]
    k1_scr[:, :] = k1s

    def _count_step(r, acc):
        km = k1_scr[pl.ds(r, 1), :]
        km3 = km.reshape(1, 1, cols)
        lt = jnp.where(km3 < kb, jnp.float32(1), jnp.float32(0))
        return acc + jnp.sum(lt, axis=2)

    cnt_f = jax.lax.fori_loop(0, rows, _count_step,
                              jnp.zeros((rows, cols), jnp.float32))
    cnt = cnt_f.astype(_I32)

    cnt_h = jax.lax.shift_right_logical(cnt, _I32(7))
    cnt_l = cnt & _I32(cols - 1)
    lane_iota3 = jax.lax.broadcasted_iota(_I32, (1, 1, cols), 2)
    oh_l = jnp.where(cnt_l[:, :, None] == lane_iota3, jnp.float32(1),
                     jnp.float32(0))
    t_mid = jax.lax.dot_general(
        oh_l.reshape(total, cols), p_ind.T,
        dimension_numbers=(((1,), (0,)), ((), ())),
        preferred_element_type=jnp.float32)
    t3 = t_mid.reshape(rows, cols, rows)
    row_iota3 = jax.lax.broadcasted_iota(_I32, (1, 1, rows), 2)
    oh_h = jnp.where(cnt_h[:, :, None] == row_iota3, jnp.float32(1),
                     jnp.float32(0))
    sel = jnp.sum(oh_h * t3, axis=2)

    ci = jax.lax.broadcasted_iota(_I32, (cols, cols), 0)
    cj = jax.lax.broadcasted_iota(_I32, (cols, cols), 1)
    d_in = cj - ci
    m_in = jnp.where((d_in >= 0) & (d_in < _I32(mask_len)), jnp.float32(1),
                     jnp.float32(0))
    d_x = cj + _I32(cols) - ci
    m_x = jnp.where((d_x >= 0) & (d_x < _I32(mask_len)), jnp.float32(1),
                    jnp.float32(0))
    hit = jax.lax.dot_general(
        sel, m_in, dimension_numbers=(((1,), (0,)), ((), ())),
        preferred_element_type=jnp.float32)
    sel_prev = jnp.concatenate(
        [jnp.zeros((1, cols), jnp.float32), sel[:rows - 1, :]], axis=0)
    hit = hit + jax.lax.dot_general(
        sel_prev, m_x, dimension_numbers=(((1,), (0,)), ((), ())),
        preferred_element_type=jnp.float32)
    o_ref[0, :, :] = (hit > jnp.float32(0)).astype(jnp.int8)


@jax.jit
def kernel(x):
    B, T, C = x.shape
    total_masked_length = int(T * _MASK_PROB)
    num_masks = math.ceil(total_masked_length / _MASK_LENGTH)
    valid_starts = T - _MASK_LENGTH + 1
    if valid_starts <= 0:
        return jnp.zeros((B, T), dtype=bool)
    n_take = min(num_masks, valid_starts)
    num_rounds = int(
        np.ceil(3 * np.log(max(1, valid_starts)) / np.log(2**32 - 1)))
    cols = 128
    rows = T // cols

    out = pl.pallas_call(
        partial(_mask_kernel, rows=rows, cols=cols, n_valid=valid_starts,
                n_take=n_take, num_rounds=num_rounds, mask_len=_MASK_LENGTH),
        grid=(B,),
        out_specs=pl.BlockSpec((1, rows, cols), lambda b: (b, 0, 0)),
        out_shape=jax.ShapeDtypeStruct((B, rows, cols), jnp.int8),
        scratch_shapes=[pltpu.VMEM((rows, cols), jnp.int32)],
    )()
    return out.reshape(B, T).astype(bool)


# speedup vs baseline: 7.0014x; 7.0014x over previous
import math
from functools import partial

import jax
import jax.numpy as jnp
import numpy as np
from jax.experimental import pallas as pl
from jax.experimental.pallas import tpu as pltpu

_MASK_PROB = 0.065
_MASK_LENGTH = 10

_I32 = jnp.int32
_F32 = jnp.float32
_SIGN = np.int32(-2147483648)


def _rotl(x, r):
    return jax.lax.shift_left(x, _I32(r)) | jax.lax.shift_right_logical(
        x, _I32(32 - r))


def _threefry2x32(k0, k1, x0, x1):
    ks0, ks1 = k0, k1
    ks2 = ks0 ^ ks1 ^ np.int32(0x1BD11BDA)
    rots = ((13, 15, 26, 6), (17, 29, 16, 24))
    sched = ((ks1, ks2), (ks2, ks0), (ks0, ks1), (ks1, ks2), (ks2, ks0))
    x0 = x0 + ks0
    x1 = x1 + ks1
    for i in range(5):
        for r in rots[i % 2]:
            x0 = x0 + x1
            x1 = _rotl(x1, r)
            x1 = x1 ^ x0
        a, b = sched[i]
        x0 = x0 + a
        x1 = x1 + b + np.int32(i + 1)
    return x0, x1


def _child_key(k0, k1, idx):
    return _threefry2x32(k0, k1, _I32(0), _I32(idx))


def _row_bits_sortable(k0, k1, lanes_iota, n_valid, pad_val):
    o0, o1 = _threefry2x32(k0, k1, jnp.zeros_like(lanes_iota), lanes_iota)
    bits = o0 ^ o1
    srt = bits ^ _SIGN
    return jnp.where(lanes_iota < _I32(n_valid), srt,
                     jnp.full_like(lanes_iota, pad_val))


def _mask_kernel(jcol_ref, o_ref, *, rows, cols, n_valid, n_take, num_rounds,
                 mask_len, nq):
    b = pl.program_id(0)
    flat_iota = (
        jax.lax.broadcasted_iota(_I32, (rows, cols), 0) * _I32(cols)
        + jax.lax.broadcasted_iota(_I32, (rows, cols), 1))
    imax = np.int32(2147483647)

    rk0, rk1 = _threefry2x32(_I32(0), _I32(42), _I32(0), b)
    round_sort_keys = []
    for _ in range(num_rounds):
        nk0, nk1 = _child_key(rk0, rk1, 0)
        sk0, sk1 = _child_key(rk0, rk1, 1)
        round_sort_keys.append(
            _row_bits_sortable(sk0, sk1, flat_iota, n_valid, imax))
        rk0, rk1 = nk0, nk1
    k1s = round_sort_keys[0]
    k2s = round_sort_keys[-1]

    def _descend54(j, v):
        bit = jax.lax.shift_left(_I32(1), _I32(31) - j)
        try_pat = v | bit
        try_s = try_pat ^ _SIGN
        cnt_lt = jnp.sum(jnp.where(k2s < try_s, _F32(1), _F32(0)))
        return jnp.where(cnt_lt >= _F32(n_take), v, try_pat)

    v54_s = jax.lax.fori_loop(0, 32, _descend54, _I32(0)) ^ _SIGN
    p_ind = jnp.where(k2s <= v54_s, _F32(1), _F32(0))

    ci = jax.lax.broadcasted_iota(_I32, (cols, cols), 0)
    cj = jax.lax.broadcasted_iota(_I32, (cols, cols), 1)
    u_strict = jnp.where(ci < cj, _F32(1), _F32(0))
    e_in = jax.lax.dot_general(p_ind, u_strict,
                               dimension_numbers=(((1,), (0,)), ((), ())),
                               preferred_element_type=_F32)
    row_sum = jnp.sum(p_ind, axis=1, keepdims=True)
    ri = jax.lax.broadcasted_iota(_I32, (rows, rows), 0)
    rj = jax.lax.broadcasted_iota(_I32, (rows, rows), 1)
    w_strict = jnp.where(rj < ri, _F32(1), _F32(0))
    row_off = jax.lax.dot_general(w_strict, row_sum,
                                  dimension_numbers=(((1,), (0,)), ((), ())),
                                  preferred_element_type=_F32)
    cpre = e_in + row_off

    t_tot = rows * cols
    c_flat = cpre.reshape(1, t_tot)
    p_flat = p_ind.reshape(1, t_tot)
    flat_f = flat_iota.astype(_F32).reshape(1, t_tot)
    j_col = jcol_ref[:, :]
    memb = (c_flat == j_col) & (p_flat > _F32(0))
    targets_col = jnp.sum(jnp.where(memb, flat_f, _F32(0)),
                          axis=1, keepdims=True)

    k1f = k1s.reshape(1, rows * cols)

    def _descend_ranks(j, v):
        bit = jax.lax.shift_left(_I32(1), _I32(31) - j)
        try_pat = v | bit
        try_s = try_pat ^ _SIGN
        ltf = jnp.where(k1f < try_s, _F32(1), _F32(0))
        cnt = jnp.sum(ltf, axis=1, keepdims=True)
        return jnp.where(cnt > targets_col, v, try_pat)

    v_pat = jax.lax.fori_loop(0, 32, _descend_ranks,
                              jnp.zeros((nq, 1), _I32))
    v_s = v_pat ^ _SIGN

    live = j_col < _F32(n_take)
    eqf = jnp.where((k1f == v_s) & live, _F32(1), _F32(0))
    self_f = jnp.sum(eqf, axis=0, keepdims=True)
    sel = self_f.reshape(rows, cols)

    d_in = cj - ci
    m_in = jnp.where((d_in >= 0) & (d_in < _I32(mask_len)), _F32(1), _F32(0))
    d_x = cj + _I32(cols) - ci
    m_x = jnp.where((d_x >= 0) & (d_x < _I32(mask_len)), _F32(1), _F32(0))
    hit = jax.lax.dot_general(sel, m_in,
                              dimension_numbers=(((1,), (0,)), ((), ())),
                              preferred_element_type=_F32)
    sel_prev = jnp.concatenate(
        [jnp.zeros((1, cols), _F32), sel[:rows - 1, :]], axis=0)
    hit = hit + jax.lax.dot_general(sel_prev, m_x,
                                    dimension_numbers=(((1,), (0,)), ((), ())),
                                    preferred_element_type=_F32)
    o_ref[0, :, :] = (hit > _F32(0)).astype(jnp.int8)


@jax.jit
def kernel(x):
    B, T, C = x.shape
    total_masked_length = int(T * _MASK_PROB)
    num_masks = math.ceil(total_masked_length / _MASK_LENGTH)
    valid_starts = T - _MASK_LENGTH + 1
    if valid_starts <= 0:
        return jnp.zeros((B, T), dtype=bool)
    n_take = min(num_masks, valid_starts)
    num_rounds = int(
        np.ceil(3 * np.log(max(1, valid_starts)) / np.log(2**32 - 1)))
    cols = 128
    rows = T // cols
    nq = ((n_take + 7) // 8) * 8

    out = pl.pallas_call(
        partial(_mask_kernel, rows=rows, cols=cols, n_valid=valid_starts,
                n_take=n_take, num_rounds=num_rounds, mask_len=_MASK_LENGTH,
                nq=nq),
        grid=(B,),
        in_specs=[pl.BlockSpec((nq, 1), lambda b: (0, 0))],
        out_specs=pl.BlockSpec((1, rows, cols), lambda b: (b, 0, 0)),
        out_shape=jax.ShapeDtypeStruct((B, rows, cols), jnp.int8),
        compiler_params=pltpu.CompilerParams(
            dimension_semantics=("parallel",)),
    )(jnp.arange(nq, dtype=jnp.float32).reshape(nq, 1))
    return out.reshape(B, T).astype(bool)


# speedup vs baseline: 8.4494x; 1.2068x over previous
import math
from functools import partial

import jax
import jax.numpy as jnp
import numpy as np
from jax.experimental import pallas as pl
from jax.experimental.pallas import tpu as pltpu

_MASK_PROB = 0.065
_MASK_LENGTH = 10

_I32 = jnp.int32
_F32 = jnp.float32
_SIGN = np.int32(-2147483648)


def _rotl(x, r):
    return jax.lax.shift_left(x, _I32(r)) | jax.lax.shift_right_logical(
        x, _I32(32 - r))


def _threefry2x32(k0, k1, x0, x1):
    ks0, ks1 = k0, k1
    ks2 = ks0 ^ ks1 ^ np.int32(0x1BD11BDA)
    rots = ((13, 15, 26, 6), (17, 29, 16, 24))
    sched = ((ks1, ks2), (ks2, ks0), (ks0, ks1), (ks1, ks2), (ks2, ks0))
    x0 = x0 + ks0
    x1 = x1 + ks1
    for i in range(5):
        for r in rots[i % 2]:
            x0 = x0 + x1
            x1 = _rotl(x1, r)
            x1 = x1 ^ x0
        a, b = sched[i]
        x0 = x0 + a
        x1 = x1 + b + np.int32(i + 1)
    return x0, x1


def _child_key(k0, k1, idx):
    return _threefry2x32(k0, k1, _I32(0), _I32(idx))


def _row_bits_sortable(k0, k1, lanes_iota, n_valid, pad_val):
    o0, o1 = _threefry2x32(k0, k1, jnp.zeros_like(lanes_iota), lanes_iota)
    bits = o0 ^ o1
    srt = bits ^ _SIGN
    return jnp.where(lanes_iota < _I32(n_valid), srt,
                     jnp.full_like(lanes_iota, pad_val))


def _bitonic_sort_pairs(key, idx, flat_iota, rows, cols, total):
    size = 2
    while size <= total:
        d = size // 2
        while d >= 1:
            if d < cols:
                fwd_k = pltpu.roll(key, cols - d, 1)
                bwd_k = pltpu.roll(key, d, 1)
                fwd_i = pltpu.roll(idx, cols - d, 1)
                bwd_i = pltpu.roll(idx, d, 1)
            else:
                dr = d // cols
                fwd_k = pltpu.roll(key, rows - dr, 0)
                bwd_k = pltpu.roll(key, dr, 0)
                fwd_i = pltpu.roll(idx, rows - dr, 0)
                bwd_i = pltpu.roll(idx, dr, 0)
            first = (flat_iota & _I32(d)) == 0
            pk = jnp.where(first, fwd_k, bwd_k)
            pi = jnp.where(first, fwd_i, bwd_i)
            want_min = first == ((flat_iota & _I32(size)) == 0)
            lt = pk < key
            gt = pk > key
            take = (want_min & lt) | (jnp.logical_not(want_min) & gt)
            key = jnp.where(take, pk, key)
            idx = jnp.where(take, pi, idx)
            d //= 2
        size *= 2
    return key, idx


def _mask_kernel(icol_ref, o_ref, *, rows, cols, n_valid, n_take, num_rounds,
                 mask_len):
    b = pl.program_id(0)
    total = rows * cols
    flat_iota = (
        jax.lax.broadcasted_iota(_I32, (rows, cols), 0) * _I32(cols)
        + jax.lax.broadcasted_iota(_I32, (rows, cols), 1))
    imax = np.int32(2147483647)

    rk0, rk1 = _threefry2x32(_I32(0), _I32(42), _I32(0), b)
    round_sort_keys = []
    for _ in range(num_rounds):
        nk0, nk1 = _child_key(rk0, rk1, 0)
        sk0, sk1 = _child_key(rk0, rk1, 1)
        round_sort_keys.append(
            _row_bits_sortable(sk0, sk1, flat_iota, n_valid, imax))
        rk0, rk1 = nk0, nk1
    k1s = round_sort_keys[0]
    k2s = round_sort_keys[-1]

    def _descend54(j, v):
        bit = jax.lax.shift_left(_I32(1), _I32(31) - j)
        try_pat = v | bit
        try_s = try_pat ^ _SIGN
        cnt_lt = jnp.sum(jnp.where(k2s < try_s, _F32(1), _F32(0)))
        return jnp.where(cnt_lt >= _F32(n_take), v, try_pat)

    v54_s = jax.lax.fori_loop(0, 32, _descend54, _I32(0)) ^ _SIGN
    p_flat = jnp.where(k2s <= v54_s, _F32(1), _F32(0)).reshape(1, total)

    _, sidx = _bitonic_sort_pairs(k1s, flat_iota, flat_iota, rows, cols, total)

    s_flat = sidx.reshape(1, total)
    vh = jax.lax.shift_right_logical(s_flat, _I32(7)).astype(_F32)
    vl = (s_flat & _I32(cols - 1)).astype(_F32)
    icol = icol_ref[:, :]
    ih_col = icol[:rows, :]
    s1t = jnp.where((vh == ih_col) & (p_flat > _F32(0)), _F32(1), _F32(0))
    s2t = jnp.where(vl == icol, _F32(1), _F32(0))
    sel = jax.lax.dot_general(
        s1t, s2t, dimension_numbers=(((1,), (1,)), ((), ())),
        preferred_element_type=_F32)

    ci = jax.lax.broadcasted_iota(_I32, (cols, cols), 0)
    cj = jax.lax.broadcasted_iota(_I32, (cols, cols), 1)
    d_in = cj - ci
    m_in = jnp.where((d_in >= 0) & (d_in < _I32(mask_len)), _F32(1), _F32(0))
    d_x = cj + _I32(cols) - ci
    m_x = jnp.where((d_x >= 0) & (d_x < _I32(mask_len)), _F32(1), _F32(0))
    hit = jax.lax.dot_general(sel, m_in,
                              dimension_numbers=(((1,), (0,)), ((), ())),
                              preferred_element_type=_F32)
    sel_prev = jnp.concatenate(
        [jnp.zeros((1, cols), _F32), sel[:rows - 1, :]], axis=0)
    hit = hit + jax.lax.dot_general(sel_prev, m_x,
                                    dimension_numbers=(((1,), (0,)), ((), ())),
                                    preferred_element_type=_F32)
    o_ref[0, :, :] = (hit > _F32(0)).astype(jnp.int8)


@jax.jit
def kernel(x):
    B, T, C = x.shape
    total_masked_length = int(T * _MASK_PROB)
    num_masks = math.ceil(total_masked_length / _MASK_LENGTH)
    valid_starts = T - _MASK_LENGTH + 1
    if valid_starts <= 0:
        return jnp.zeros((B, T), dtype=bool)
    n_take = min(num_masks, valid_starts)
    num_rounds = int(
        np.ceil(3 * np.log(max(1, valid_starts)) / np.log(2**32 - 1)))
    cols = 128
    rows = T // cols

    out = pl.pallas_call(
        partial(_mask_kernel, rows=rows, cols=cols, n_valid=valid_starts,
                n_take=n_take, num_rounds=num_rounds, mask_len=_MASK_LENGTH),
        grid=(B,),
        in_specs=[pl.BlockSpec((cols, 1), lambda b: (0, 0))],
        out_specs=pl.BlockSpec((1, rows, cols), lambda b: (b, 0, 0)),
        out_shape=jax.ShapeDtypeStruct((B, rows, cols), jnp.int8),
        compiler_params=pltpu.CompilerParams(
            dimension_semantics=("parallel",)),
    )(jnp.arange(cols, dtype=jnp.float32).reshape(cols, 1))
    return out.reshape(B, T).astype(bool)


# speedup vs baseline: 16.7492x; 1.9823x over previous
import math
from functools import partial

import jax
import jax.numpy as jnp
import numpy as np
from jax.experimental import pallas as pl
from jax.experimental.pallas import tpu as pltpu

_MASK_PROB = 0.065
_MASK_LENGTH = 10

_I32 = jnp.int32
_F32 = jnp.float32
_SIGN = np.int32(-2147483648)


def _rotl(x, r):
    return jax.lax.shift_left(x, _I32(r)) | jax.lax.shift_right_logical(
        x, _I32(32 - r))


def _threefry2x32(k0, k1, x0, x1):
    ks0, ks1 = k0, k1
    ks2 = ks0 ^ ks1 ^ np.int32(0x1BD11BDA)
    rots = ((13, 15, 26, 6), (17, 29, 16, 24))
    sched = ((ks1, ks2), (ks2, ks0), (ks0, ks1), (ks1, ks2), (ks2, ks0))
    x0 = x0 + ks0
    x1 = x1 + ks1
    for i in range(5):
        for r in rots[i % 2]:
            x0 = x0 + x1
            x1 = _rotl(x1, r)
            x1 = x1 ^ x0
        a, b = sched[i]
        x0 = x0 + a
        x1 = x1 + b + np.int32(i + 1)
    return x0, x1


def _bitonic_sort_pairs(key, idx, flat_iota, rows, cols, total):
    size = 2
    while size <= total:
        d = size // 2
        while d >= 1:
            if d < cols:
                fwd_k = pltpu.roll(key, cols - d, 2)
                bwd_k = pltpu.roll(key, d, 2)
                fwd_i = pltpu.roll(idx, cols - d, 2)
                bwd_i = pltpu.roll(idx, d, 2)
            else:
                dr = d // cols
                fwd_k = pltpu.roll(key, rows - dr, 1)
                bwd_k = pltpu.roll(key, dr, 1)
                fwd_i = pltpu.roll(idx, rows - dr, 1)
                bwd_i = pltpu.roll(idx, dr, 1)
            first = (flat_iota & _I32(d)) == 0
            pk = jnp.where(first, fwd_k, bwd_k)
            pi = jnp.where(first, fwd_i, bwd_i)
            want_min = first == ((flat_iota & _I32(size)) == 0)
            lt = pk < key
            gt = pk > key
            take = (want_min & lt) | (jnp.logical_not(want_min) & gt)
            key = jnp.where(take, pk, key)
            idx = jnp.where(take, pi, idx)
            d //= 2
        size *= 2
    return key, idx


def _mask_kernel(icol_ref, o_ref, *, nb, rows, cols, n_valid, n_take,
                 num_rounds, mask_len):
    total = rows * cols
    flat_iota = (
        jax.lax.broadcasted_iota(_I32, (nb, rows, cols), 1) * _I32(cols)
        + jax.lax.broadcasted_iota(_I32, (nb, rows, cols), 2))
    brow = jax.lax.broadcasted_iota(_I32, (nb, rows, cols), 0)
    imax = np.int32(2147483647)

    rk0, rk1 = _threefry2x32(_I32(0), _I32(42), _I32(0), brow)
    zero = jnp.zeros_like(flat_iota)
    round_sort_keys = []
    for _ in range(num_rounds):
        nk0, nk1 = _threefry2x32(rk0, rk1, zero, zero)
        sk0, sk1 = _threefry2x32(rk0, rk1, zero, jnp.ones_like(flat_iota))
        o0, o1 = _threefry2x32(sk0, sk1, zero, flat_iota)
        srt = (o0 ^ o1) ^ _SIGN
        round_sort_keys.append(
            jnp.where(flat_iota < _I32(n_valid), srt,
                      jnp.full_like(flat_iota, imax)))
        rk0, rk1 = nk0, nk1
    k1s = round_sort_keys[0]
    k2s = round_sort_keys[-1]

    def _descend54(j, v):
        bit = jax.lax.shift_left(_I32(1), _I32(31) - j)
        try_pat = v | bit
        try_s = try_pat ^ _SIGN
        cnt = jnp.sum(jnp.where(k2s < try_s, _F32(1), _F32(0)),
                      axis=(1, 2), keepdims=True)
        return jnp.where(cnt >= _F32(n_take), v, try_pat)

    v54 = jax.lax.fori_loop(0, 32, _descend54,
                            jnp.zeros((nb, rows, cols), _I32))
    v54_s = v54 ^ _SIGN
    p_ind = jnp.where(k2s <= v54_s, _F32(1), _F32(0))

    _, sidx = _bitonic_sort_pairs(k1s, flat_iota, flat_iota, rows, cols, total)

    ci = jax.lax.broadcasted_iota(_I32, (cols, cols), 0)
    cj = jax.lax.broadcasted_iota(_I32, (cols, cols), 1)
    d_in = cj - ci
    m_in = jnp.where((d_in >= 0) & (d_in < _I32(mask_len)), _F32(1), _F32(0))
    d_x = cj + _I32(cols) - ci
    m_x = jnp.where((d_x >= 0) & (d_x < _I32(mask_len)), _F32(1), _F32(0))
    icol = icol_ref[:, :]
    ih_col = icol[:rows, :]

    for r in range(nb):
        s_flat = sidx[r].reshape(1, total)
        p_flat = p_ind[r].reshape(1, total)
        vh = jax.lax.shift_right_logical(s_flat, _I32(7)).astype(_F32)
        vl = (s_flat & _I32(cols - 1)).astype(_F32)
        s1t = jnp.where((vh == ih_col) & (p_flat > _F32(0)), _F32(1), _F32(0))
        s2t = jnp.where(vl == icol, _F32(1), _F32(0))
        sel = jax.lax.dot_general(
            s1t, s2t, dimension_numbers=(((1,), (1,)), ((), ())),
            preferred_element_type=_F32)
        hit = jax.lax.dot_general(sel, m_in,
                                  dimension_numbers=(((1,), (0,)), ((), ())),
                                  preferred_element_type=_F32)
        sel_prev = jnp.concatenate(
            [jnp.zeros((1, cols), _F32), sel[:rows - 1, :]], axis=0)
        hit = hit + jax.lax.dot_general(
            sel_prev, m_x, dimension_numbers=(((1,), (0,)), ((), ())),
            preferred_element_type=_F32)
        o_ref[r, :, :] = (hit > _F32(0)).astype(jnp.int8)


@jax.jit
def kernel(x):
    B, T, C = x.shape
    total_masked_length = int(T * _MASK_PROB)
    num_masks = math.ceil(total_masked_length / _MASK_LENGTH)
    valid_starts = T - _MASK_LENGTH + 1
    if valid_starts <= 0:
        return jnp.zeros((B, T), dtype=bool)
    n_take = min(num_masks, valid_starts)
    num_rounds = int(
        np.ceil(3 * np.log(max(1, valid_starts)) / np.log(2**32 - 1)))
    cols = 128
    rows = T // cols

    out = pl.pallas_call(
        partial(_mask_kernel, nb=B, rows=rows, cols=cols, n_valid=valid_starts,
                n_take=n_take, num_rounds=num_rounds, mask_len=_MASK_LENGTH),
        grid=(1,),
        in_specs=[pl.BlockSpec((cols, 1), lambda b: (0, 0))],
        out_specs=pl.BlockSpec((B, rows, cols), lambda b: (0, 0, 0)),
        out_shape=jax.ShapeDtypeStruct((B, rows, cols), jnp.int8),
    )(jnp.arange(cols, dtype=jnp.float32).reshape(cols, 1))
    return out.reshape(B, T).astype(bool)
